# Initial kernel scaffold; baseline (speedup 1.0000x reference)
import jax
import jax.numpy as jnp
from jax.experimental import pallas as pl


def kernel(x, table, W, b):
    raise NotImplementedError("write your pallas kernel here")


# speedup vs baseline: 7.2344x; 7.2344x over previous
import functools

import jax
import jax.numpy as jnp
from jax import lax
from jax.experimental import pallas as pl
from jax.experimental.pallas import tpu as pltpu
from jax.experimental.pallas import tpu_sc as plsc

VOCAB = 100000
EMBED = 100
EP = 112
SEQ = 260
BATCH = 4096
NC = 2
NS = 16
NW = NC * NS
BPW = BATCH // NW

_NCHUNK = EP // 16


def _sc_body(xT_hbm, tbl_hbm, wt_hbm, out_hbm,
             w_buf, xblk, rows0, rows1, accv,
             sem_w, sem_x, sg0, sg1):
  cid = lax.axis_index("c")
  sid = lax.axis_index("s")
  wid = sid * NC + cid
  base = wid * BPW

  cp_w = pltpu.async_copy(wt_hbm, w_buf, sem_w)
  cp_x = pltpu.async_copy(xT_hbm.at[:, pl.ds(base, BPW)], xblk, sem_x)

  zero = jnp.zeros((16,), jnp.float32)

  def zbody(i, _):
    accv[i, pl.ds(0, 16)] = zero
    accv[i, pl.ds(16, 16)] = zero
    return 0

  lax.fori_loop(0, BPW, zbody, 0)

  cp_x.wait()
  pltpu.async_copy(tbl_hbm.at[xblk.at[0]], rows0, sg0)
  cp_w.wait()

  def compute(l, rows):
    ws0 = [w_buf[l, 0, pl.ds(k * 16, 16)] for k in range(_NCHUNK)]
    ws1 = [w_buf[l, 1, pl.ds(k * 16, 16)] for k in range(_NCHUNK)]

    def bbody(bb, _):
      a0 = accv[bb, pl.ds(0, 16)]
      a1 = accv[bb, pl.ds(16, 16)]
      for k in range(_NCHUNK):
        r = rows[bb, pl.ds(k * 16, 16)]
        a0 = a0 + r * ws0[k]
        a1 = a1 + r * ws1[k]
      accv[bb, pl.ds(0, 16)] = a0
      accv[bb, pl.ds(16, 16)] = a1
      return 0

    lax.fori_loop(0, BPW, bbody, 0)

  def tbody(t, _):
    l0 = 2 * t
    l1 = l0 + 1
    pltpu.async_copy(tbl_hbm.at[xblk.at[l1]], rows1, sg1)
    pltpu.make_async_copy(tbl_hbm.at[xblk.at[l0]], rows0, sg0).wait()
    compute(l0, rows0)

    @pl.when(t < SEQ // 2 - 1)
    def _():
      pltpu.async_copy(tbl_hbm.at[xblk.at[l0 + 2]], rows0, sg0)

    pltpu.make_async_copy(tbl_hbm.at[xblk.at[l1]], rows1, sg1).wait()
    compute(l1, rows1)
    return 0

  lax.fori_loop(0, SEQ // 2, tbody, 0)
  pltpu.sync_copy(accv, out_hbm.at[pl.ds(base, BPW), :])


_sc_partial = functools.partial(
    pl.kernel,
    out_type=jax.ShapeDtypeStruct((BATCH, 32), jnp.float32),
    mesh=plsc.VectorSubcoreMesh(
        core_axis_name="c", subcore_axis_name="s",
        num_cores=NC, num_subcores=NS),
    compiler_params=pltpu.CompilerParams(use_tc_tiling_on_sc=False),
    scratch_types=[
        pltpu.VMEM((SEQ, 2, EP), jnp.float32),
        pltpu.VMEM((SEQ, BPW), jnp.int32),
        pltpu.VMEM((BPW, EP), jnp.float32),
        pltpu.VMEM((BPW, EP), jnp.float32),
        pltpu.VMEM((BPW, 32), jnp.float32),
        pltpu.SemaphoreType.DMA,
        pltpu.SemaphoreType.DMA,
        pltpu.SemaphoreType.DMA,
        pltpu.SemaphoreType.DMA,
    ],
)(_sc_body)


def _tc_epilogue(b_ref, p_ref, o_ref):
  blk = p_ref[...]
  s0 = jnp.sum(blk[:, 0:16], axis=1) + b_ref[0]
  s1 = jnp.sum(blk[:, 16:32], axis=1) + b_ref[1]
  m = jnp.maximum(s0, s1)
  lse = m + jnp.log(jnp.exp(s0 - m) + jnp.exp(s1 - m))
  o_ref[...] = jnp.concatenate(
      [(s0 - lse)[:, None], (s1 - lse)[:, None]], axis=1)


@jax.jit
def kernel(x, table, W, b):
  x = x.astype(jnp.int32)
  xT = x.T
  tbl_pad = jnp.pad(table, ((0, 0), (0, EP - EMBED)))

  w3 = W.reshape(SEQ, EMBED, 2).transpose(0, 2, 1)
  wt = jnp.zeros((SEQ, 2, EP), jnp.float32)
  wt = wt.at[:, :, :EMBED].set(w3)

  partial = _sc_partial(xT, tbl_pad, wt)

  blk = 512
  out = pl.pallas_call(
      _tc_epilogue,
      grid=(BATCH // blk,),
      in_specs=[
          pl.BlockSpec(memory_space=pltpu.SMEM),
          pl.BlockSpec((blk, 32), lambda i: (i, 0)),
      ],
      out_specs=pl.BlockSpec((blk, 2), lambda i: (i, 0)),
      out_shape=jax.ShapeDtypeStruct((BATCH, 2), jnp.float32),
  )(b, partial)
  return out
